# baseline (speedup 1.0000x reference)
import jax
import jax.numpy as jnp
from jax import lax
from jax.experimental import pallas as pl
from jax.experimental.pallas import tpu as pltpu

_VOCAB = 100000
_EMB = 64
_CTX = 20
_HIDDEN = 128

_BV = 5000
_NB = _VOCAB // _BV


def _body(idx_s, emb_any, w1_ref, b1_ref, w2_ref, b2_ref, out_ref, lp_ref,
          x_ref, h_ref, m_ref, sem):
    i = pl.program_id(0)

    @pl.when(i == 0)
    def _():
        copies = [
            pltpu.make_async_copy(
                emb_any.at[pl.ds(idx_s[j], 1), :],
                x_ref.at[pl.ds(j, 1), :], sem)
            for j in range(_CTX)
        ]
        for c in copies:
            c.start()
        for c in copies:
            c.wait()
        acc = b1_ref[...]
        for t in range(_CTX):
            acc = acc + lax.dot_general(
                x_ref[pl.ds(t, 1), :], w1_ref[:, pl.ds(t * _EMB, _EMB)],
                (((1,), (1,)), ((), ())),
                preferred_element_type=jnp.float32)
        h_ref[...] = jnp.maximum(acc, 0.0)

    o = lax.dot_general(
        h_ref[...], w2_ref[...], (((1,), (1,)), ((), ())),
        preferred_element_type=jnp.float32)
    o = o + b2_ref[0]
    out_ref[pl.ds(i, 1), :] = o

    bm = jnp.max(o, axis=1, keepdims=True)

    @pl.when(i == 0)
    def _():
        m_ref[...] = bm

    @pl.when(i > 0)
    def _():
        m_ref[...] = jnp.maximum(m_ref[...], bm)

    @pl.when(i == _NB - 1)
    def _():
        logits = out_ref[...]
        m = m_ref[...]
        e = jnp.exp(logits - m)
        s = jnp.sum(jnp.sum(e, axis=1, keepdims=True), axis=0, keepdims=True)
        lse = m + jnp.log(s)
        lp_ref[...] = logits - lse


def _fused(idx, emb, W1, b1r, W2, b2r):
    return pl.pallas_call(
        _body,
        grid=(_NB,),
        in_specs=[
            pl.BlockSpec(memory_space=pltpu.MemorySpace.SMEM),
            pl.BlockSpec(memory_space=pl.ANY),
            pl.BlockSpec((_HIDDEN, _CTX * _EMB), lambda i: (0, 0)),
            pl.BlockSpec((1, _HIDDEN), lambda i: (0, 0)),
            pl.BlockSpec((_BV, _HIDDEN), lambda i: (i, 0)),
            pl.BlockSpec((1, 1, _BV), lambda i: (i, 0, 0)),
        ],
        out_specs=[
            pl.BlockSpec((_NB, _BV), lambda i: (0, 0)),
            pl.BlockSpec((_NB, _BV), lambda i: (0, 0)),
        ],
        out_shape=[
            jax.ShapeDtypeStruct((_NB, _BV), jnp.float32),
            jax.ShapeDtypeStruct((_NB, _BV), jnp.float32),
        ],
        scratch_shapes=[
            pltpu.VMEM((_CTX, _EMB), jnp.float32),
            pltpu.VMEM((1, _HIDDEN), jnp.float32),
            pltpu.VMEM((1, 1), jnp.float32),
            pltpu.SemaphoreType.DMA,
        ],
    )(idx, emb, W1, b1r, W2, b2r)


def kernel(inputs, emb, W1, b1, W2, b2):
    out2d, lp2d = _fused(inputs.astype(jnp.int32), emb, W1,
                         b1.reshape(1, _HIDDEN), W2, b2.reshape(_NB, 1, _BV))
    return (out2d.reshape(1, _VOCAB), lp2d.reshape(1, _VOCAB))


# speedup vs baseline: 1.0683x; 1.0683x over previous
import jax
import jax.numpy as jnp
from jax import lax
from jax.experimental import pallas as pl
from jax.experimental.pallas import tpu as pltpu

_VOCAB = 100000
_EMB = 64
_CTX = 20
_HIDDEN = 128

_BV = 10000
_NB = _VOCAB // _BV


def _body(idx_s, emb_any, w1_ref, b1_ref, w2_ref, b2_ref, out_ref, lp_ref,
          x_ref, h_ref, m_ref, sem):
    i = pl.program_id(0)

    @pl.when(i == 0)
    def _():
        copies = [
            pltpu.make_async_copy(
                emb_any.at[pl.ds(idx_s[j], 1), :],
                x_ref.at[pl.ds(j, 1), :], sem)
            for j in range(_CTX)
        ]
        for c in copies:
            c.start()
        for c in copies:
            c.wait()
        acc = b1_ref[...]
        for t in range(_CTX):
            acc = acc + lax.dot_general(
                x_ref[pl.ds(t, 1), :], w1_ref[:, pl.ds(t * _EMB, _EMB)],
                (((1,), (1,)), ((), ())),
                preferred_element_type=jnp.float32)
        h_ref[...] = jnp.maximum(acc, 0.0)

    o = lax.dot_general(
        h_ref[...], w2_ref[...], (((1,), (1,)), ((), ())),
        preferred_element_type=jnp.float32)
    o = o + b2_ref[0]
    out_ref[pl.ds(i, 1), :] = o

    bm = jnp.max(o, axis=1, keepdims=True)

    @pl.when(i == 0)
    def _():
        m_ref[...] = bm

    @pl.when(i > 0)
    def _():
        m_ref[...] = jnp.maximum(m_ref[...], bm)

    @pl.when(i == _NB - 1)
    def _():
        logits = out_ref[...]
        m = m_ref[...]
        e = jnp.exp(logits - m)
        s = jnp.sum(jnp.sum(e, axis=1, keepdims=True), axis=0, keepdims=True)
        lse = m + jnp.log(s)
        lp_ref[...] = logits - lse


def _fused(idx, emb, W1, b1r, W2, b2r):
    return pl.pallas_call(
        _body,
        grid=(_NB,),
        in_specs=[
            pl.BlockSpec(memory_space=pltpu.MemorySpace.SMEM),
            pl.BlockSpec(memory_space=pl.ANY),
            pl.BlockSpec((_HIDDEN, _CTX * _EMB), lambda i: (0, 0)),
            pl.BlockSpec((1, _HIDDEN), lambda i: (0, 0)),
            pl.BlockSpec((_BV, _HIDDEN), lambda i: (i, 0)),
            pl.BlockSpec((1, 1, _BV), lambda i: (i, 0, 0)),
        ],
        out_specs=[
            pl.BlockSpec((_NB, _BV), lambda i: (0, 0)),
            pl.BlockSpec((_NB, _BV), lambda i: (0, 0)),
        ],
        out_shape=[
            jax.ShapeDtypeStruct((_NB, _BV), jnp.float32),
            jax.ShapeDtypeStruct((_NB, _BV), jnp.float32),
        ],
        scratch_shapes=[
            pltpu.VMEM((_CTX, _EMB), jnp.float32),
            pltpu.VMEM((1, _HIDDEN), jnp.float32),
            pltpu.VMEM((1, 1), jnp.float32),
            pltpu.SemaphoreType.DMA,
        ],
    )(idx, emb, W1, b1r, W2, b2r)


def kernel(inputs, emb, W1, b1, W2, b2):
    out2d, lp2d = _fused(inputs.astype(jnp.int32), emb, W1,
                         b1.reshape(1, _HIDDEN), W2, b2.reshape(_NB, 1, _BV))
    return (out2d.reshape(1, _VOCAB), lp2d.reshape(1, _VOCAB))


# speedup vs baseline: 1.0728x; 1.0043x over previous
import jax
import jax.numpy as jnp
from jax import lax
from jax.experimental import pallas as pl
from jax.experimental.pallas import tpu as pltpu

_VOCAB = 100000
_EMB = 64
_CTX = 20
_HIDDEN = 128

_BV = 20000
_NB = _VOCAB // _BV


def _body(idx_s, emb_any, w1_ref, b1_ref, w2_ref, b2_ref, out_ref, lp_ref,
          x_ref, h_ref, m_ref, sem):
    i = pl.program_id(0)

    @pl.when(i == 0)
    def _():
        copies = [
            pltpu.make_async_copy(
                emb_any.at[pl.ds(idx_s[j], 1), :],
                x_ref.at[pl.ds(j, 1), :], sem)
            for j in range(_CTX)
        ]
        for c in copies:
            c.start()
        for c in copies:
            c.wait()
        acc = b1_ref[...]
        for t in range(_CTX):
            acc = acc + lax.dot_general(
                x_ref[pl.ds(t, 1), :], w1_ref[:, pl.ds(t * _EMB, _EMB)],
                (((1,), (1,)), ((), ())),
                preferred_element_type=jnp.float32)
        h_ref[...] = jnp.maximum(acc, 0.0)

    o = lax.dot_general(
        h_ref[...], w2_ref[...], (((1,), (1,)), ((), ())),
        preferred_element_type=jnp.float32)
    o = o + b2_ref[0]
    out_ref[pl.ds(i, 1), :] = o

    bm = jnp.max(o, axis=1, keepdims=True)

    @pl.when(i == 0)
    def _():
        m_ref[...] = bm

    @pl.when(i > 0)
    def _():
        m_ref[...] = jnp.maximum(m_ref[...], bm)

    @pl.when(i == _NB - 1)
    def _():
        logits = out_ref[...]
        m = m_ref[...]
        e = jnp.exp(logits - m)
        s = jnp.sum(jnp.sum(e, axis=1, keepdims=True), axis=0, keepdims=True)
        lse = m + jnp.log(s)
        lp_ref[...] = logits - lse


def _fused(idx, emb, W1, b1r, W2, b2r):
    return pl.pallas_call(
        _body,
        grid=(_NB,),
        in_specs=[
            pl.BlockSpec(memory_space=pltpu.MemorySpace.SMEM),
            pl.BlockSpec(memory_space=pl.ANY),
            pl.BlockSpec((_HIDDEN, _CTX * _EMB), lambda i: (0, 0)),
            pl.BlockSpec((1, _HIDDEN), lambda i: (0, 0)),
            pl.BlockSpec((_BV, _HIDDEN), lambda i: (i, 0)),
            pl.BlockSpec((1, 1, _BV), lambda i: (i, 0, 0)),
        ],
        out_specs=[
            pl.BlockSpec((_NB, _BV), lambda i: (0, 0)),
            pl.BlockSpec((_NB, _BV), lambda i: (0, 0)),
        ],
        out_shape=[
            jax.ShapeDtypeStruct((_NB, _BV), jnp.float32),
            jax.ShapeDtypeStruct((_NB, _BV), jnp.float32),
        ],
        scratch_shapes=[
            pltpu.VMEM((_CTX, _EMB), jnp.float32),
            pltpu.VMEM((1, _HIDDEN), jnp.float32),
            pltpu.VMEM((1, 1), jnp.float32),
            pltpu.SemaphoreType.DMA,
        ],
    )(idx, emb, W1, b1r, W2, b2r)


def kernel(inputs, emb, W1, b1, W2, b2):
    out2d, lp2d = _fused(inputs.astype(jnp.int32), emb, W1,
                         b1.reshape(1, _HIDDEN), W2, b2.reshape(_NB, 1, _BV))
    return (out2d.reshape(1, _VOCAB), lp2d.reshape(1, _VOCAB))


# speedup vs baseline: 1.0774x; 1.0042x over previous
import jax
import jax.numpy as jnp
from jax import lax
from jax.experimental import pallas as pl
from jax.experimental.pallas import tpu as pltpu

_VOCAB = 100000
_EMB = 64
_CTX = 20
_HIDDEN = 128

_BV = 25000
_NB = _VOCAB // _BV


def _body(idx_s, emb_any, w1_ref, b1_ref, w2_ref, b2_ref, out_ref, lp_ref,
          x_ref, h_ref, m_ref, sem):
    i = pl.program_id(0)

    @pl.when(i == 0)
    def _():
        copies = [
            pltpu.make_async_copy(
                emb_any.at[pl.ds(idx_s[j], 1), :],
                x_ref.at[pl.ds(j, 1), :], sem)
            for j in range(_CTX)
        ]
        for c in copies:
            c.start()
        for c in copies:
            c.wait()
        acc = b1_ref[...]
        for t in range(_CTX):
            acc = acc + lax.dot_general(
                x_ref[pl.ds(t, 1), :], w1_ref[:, pl.ds(t * _EMB, _EMB)],
                (((1,), (1,)), ((), ())),
                preferred_element_type=jnp.float32)
        h_ref[...] = jnp.maximum(acc, 0.0)

    o = lax.dot_general(
        h_ref[...], w2_ref[...], (((1,), (1,)), ((), ())),
        preferred_element_type=jnp.float32)
    o = o + b2_ref[0]
    out_ref[pl.ds(i, 1), :] = o

    bm = jnp.max(o, axis=1, keepdims=True)

    @pl.when(i == 0)
    def _():
        m_ref[...] = bm

    @pl.when(i > 0)
    def _():
        m_ref[...] = jnp.maximum(m_ref[...], bm)

    @pl.when(i == _NB - 1)
    def _():
        logits = out_ref[...]
        m = m_ref[...]
        e = jnp.exp(logits - m)
        s = jnp.sum(jnp.sum(e, axis=1, keepdims=True), axis=0, keepdims=True)
        lse = m + jnp.log(s)
        lp_ref[...] = logits - lse


def _fused(idx, emb, W1, b1r, W2, b2r):
    return pl.pallas_call(
        _body,
        grid=(_NB,),
        in_specs=[
            pl.BlockSpec(memory_space=pltpu.MemorySpace.SMEM),
            pl.BlockSpec(memory_space=pl.ANY),
            pl.BlockSpec((_HIDDEN, _CTX * _EMB), lambda i: (0, 0)),
            pl.BlockSpec((1, _HIDDEN), lambda i: (0, 0)),
            pl.BlockSpec((_BV, _HIDDEN), lambda i: (i, 0)),
            pl.BlockSpec((1, 1, _BV), lambda i: (i, 0, 0)),
        ],
        out_specs=[
            pl.BlockSpec((_NB, _BV), lambda i: (0, 0)),
            pl.BlockSpec((_NB, _BV), lambda i: (0, 0)),
        ],
        out_shape=[
            jax.ShapeDtypeStruct((_NB, _BV), jnp.float32),
            jax.ShapeDtypeStruct((_NB, _BV), jnp.float32),
        ],
        scratch_shapes=[
            pltpu.VMEM((_CTX, _EMB), jnp.float32),
            pltpu.VMEM((1, _HIDDEN), jnp.float32),
            pltpu.VMEM((1, 1), jnp.float32),
            pltpu.SemaphoreType.DMA,
        ],
    )(idx, emb, W1, b1r, W2, b2r)


def kernel(inputs, emb, W1, b1, W2, b2):
    out2d, lp2d = _fused(inputs.astype(jnp.int32), emb, W1,
                         b1.reshape(1, _HIDDEN), W2, b2.reshape(_NB, 1, _BV))
    return (out2d.reshape(1, _VOCAB), lp2d.reshape(1, _VOCAB))
